# baseline (speedup 1.0000x reference)
import functools
import jax
import jax.numpy as jnp
from jax import lax
from jax.experimental import pallas as pl
from jax.experimental.pallas import tpu as pltpu
from jax.experimental.pallas import tpu_sc as plsc

F32 = jnp.float32
BF16 = jnp.bfloat16
NW = 32
SUB = 40


def _silu(x):
    return x * jax.nn.sigmoid(x)


def _tc_prep(nfp, fcp, W1a, W1b, lat9, W1cp):
    NP, H = nfp.shape
    B = lat9.shape[0]
    BLK = 512
    WID = H + 32

    def body(nf_ref, fc_ref, wa_ref, wb_ref, l_ref, wc_ref,
             ts_ref, td_ref, lp_ref):
        nf = nf_ref[...]
        fc = fc_ref[...]
        ts_ref[:, :H] = jnp.dot(
            nf, wa_ref[...], preferred_element_type=F32).astype(BF16)
        ts_ref[:, H:] = (-fc).astype(BF16)
        td_ref[:, :H] = jnp.dot(
            nf, wb_ref[...], preferred_element_type=F32).astype(BF16)
        td_ref[:, H:] = fc.astype(BF16)
        l = l_ref[...]
        cols = []
        for i in range(3):
            for j in range(3):
                v = (l[:, 3 * i + 0:3 * i + 1] * l[:, 3 * j + 0:3 * j + 1]
                     + l[:, 3 * i + 1:3 * i + 2] * l[:, 3 * j + 1:3 * j + 2]
                     + l[:, 3 * i + 2:3 * i + 3] * l[:, 3 * j + 2:3 * j + 3])
                cols.append(v)
        cols.append(jnp.zeros_like(l[:, :7]))
        llt = jnp.concatenate(cols, axis=1)
        lp_ref[...] = jnp.dot(llt, wc_ref[...], preferred_element_type=F32)

    return pl.pallas_call(
        body,
        grid=(NP // BLK,),
        in_specs=[
            pl.BlockSpec((BLK, H), lambda i: (i, 0)),
            pl.BlockSpec((BLK, 32), lambda i: (i, 0)),
            pl.BlockSpec((H, H), lambda i: (0, 0)),
            pl.BlockSpec((H, H), lambda i: (0, 0)),
            pl.BlockSpec((B, 16), lambda i: (0, 0)),
            pl.BlockSpec((16, H), lambda i: (0, 0)),
        ],
        out_specs=[
            pl.BlockSpec((BLK, WID), lambda i: (i, 0)),
            pl.BlockSpec((BLK, WID), lambda i: (i, 0)),
            pl.BlockSpec((B, H), lambda i: (0, 0)),
        ],
        out_shape=[
            jax.ShapeDtypeStruct((NP, WID), BF16),
            jax.ShapeDtypeStruct((NP, WID), BF16),
            jax.ShapeDtypeStruct((B, H), F32),
        ],
    )(nfp, fcp, W1a, W1b, lat9, W1cp)


def _sc_gather(TS, TD, src2, dst2, E):
    W = TS.shape[1]
    EW = E // NW
    CH = EW // SUB
    mesh = plsc.VectorSubcoreMesh(core_axis_name="c", subcore_axis_name="s")

    H = 128

    @functools.partial(
        pl.kernel,
        mesh=mesh,
        out_type=(jax.ShapeDtypeStruct((E, H), BF16),
                  jax.ShapeDtypeStruct((E, 32), BF16)),
        compiler_params=pltpu.CompilerParams(use_tc_tiling_on_sc=False),
        scratch_types=[
            pltpu.VMEM((CH, SUB), jnp.int32),
            pltpu.VMEM((CH, SUB), jnp.int32),
            pltpu.VMEM((SUB, W), BF16),
            pltpu.VMEM((SUB, W), BF16),
            pltpu.VMEM((SUB, W), BF16),
            pltpu.VMEM((SUB, W), BF16),
            pltpu.VMEM((SUB, H), BF16),
            pltpu.VMEM((SUB, H), BF16),
            pltpu.VMEM((SUB, 32), BF16),
            pltpu.VMEM((SUB, 32), BF16),
            pltpu.SemaphoreType.DMA,
            pltpu.SemaphoreType.DMA,
            pltpu.SemaphoreType.DMA,
            pltpu.SemaphoreType.DMA,
            pltpu.SemaphoreType.DMA,
            pltpu.SemaphoreType.DMA,
            pltpu.SemaphoreType.DMA,
            pltpu.SemaphoreType.DMA,
        ],
    )
    def k(ts_hbm, td_hbm, src_hbm, dst_hbm, g0_hbm, d_hbm,
          idxs, idxd, bufa0, bufa1, bufb0, bufb1, gbuf0, gbuf1, dbuf0, dbuf1,
          la0, la1, lb0, lb1, sg0, sg1, sd0, sd1):
        cid = lax.axis_index("c")
        sid = lax.axis_index("s")
        w32 = cid * 16 + sid
        pltpu.sync_copy(src_hbm.at[w32], idxs)
        pltpu.sync_copy(dst_hbm.at[w32], idxd)

        bufa = (bufa0, bufa1)
        bufb = (bufb0, bufb1)
        gbuf = (gbuf0, gbuf1)
        dbuf = (dbuf0, dbuf1)
        la = (la0, la1)
        lb = (lb0, lb1)
        sg = (sg0, sg1)
        sd = (sd0, sd1)

        pltpu.async_copy(ts_hbm.at[idxs.at[0]], bufa0, la0)
        pltpu.async_copy(td_hbm.at[idxd.at[0]], bufb0, lb0)

        def pair(i, carry):
            for b in range(2):
                j = 2 * i + b
                o = 1 - b

                @pl.when(j + 1 < CH)
                def _():
                    pltpu.async_copy(ts_hbm.at[idxs.at[j + 1]], bufa[o], la[o])
                    pltpu.async_copy(td_hbm.at[idxd.at[j + 1]], bufb[o], lb[o])

                pltpu.make_async_copy(
                    ts_hbm.at[pl.ds(0, SUB)], bufa[b], la[b]).wait()
                pltpu.make_async_copy(
                    td_hbm.at[pl.ds(0, SUB)], bufb[b], lb[b]).wait()

                @pl.when(j >= 2)
                def _():
                    pltpu.make_async_copy(
                        g0_hbm.at[pl.ds(0, SUB)], gbuf[b], sg[b]).wait()
                    pltpu.make_async_copy(
                        d_hbm.at[pl.ds(0, SUB)], dbuf[b], sd[b]).wait()

                def radd(r, c2, _b=b):
                    for cc in range(H // 32):
                        sl = pl.ds(cc * 32, 32)
                        gbuf[_b][r, sl] = bufa[_b][r, sl] + bufb[_b][r, sl]
                    sl = pl.ds(H, 32)
                    dbuf[_b][r, :] = bufa[_b][r, sl] + bufb[_b][r, sl]
                    return c2

                lax.fori_loop(0, SUB, radd, 0)
                rows = pl.ds(w32 * EW + j * SUB, SUB)
                pltpu.async_copy(gbuf[b], g0_hbm.at[rows], sg[b])
                pltpu.async_copy(dbuf[b], d_hbm.at[rows], sd[b])
            return carry

        lax.fori_loop(0, CH // 2, pair, 0)
        for b in range(2):
            pltpu.make_async_copy(
                g0_hbm.at[pl.ds(0, SUB)], gbuf[b], sg[b]).wait()
            pltpu.make_async_copy(
                d_hbm.at[pl.ds(0, SUB)], dbuf[b], sd[b]).wait()

    return k(TS, TD, src2, dst2)


def _tc_edge(G0, D, e2g3, latp, Wfp, W2, b1r, b2r):
    E, H = G0.shape
    B = latp.shape[0]
    BLK = 512

    def body(g_ref, d_ref, id_ref, lp_ref, wf_ref, w2_ref, b1_ref, b2_ref,
             out_ref):
        g = g_ref[...].astype(F32)
        d = d_ref[...].astype(F32)
        fd = d - jnp.floor(d)
        ids = id_ref[0, 0, :]
        oh = (ids[:, None] == lax.broadcasted_iota(jnp.int32, (BLK, B), 1)
              ).astype(F32)
        pre = (g
               + jnp.dot(oh, lp_ref[...], preferred_element_type=F32)
               + jnp.dot(fd, wf_ref[...], preferred_element_type=F32)
               + b1_ref[...])
        e1 = _silu(pre)
        out_ref[...] = _silu(
            jnp.dot(e1, w2_ref[...], preferred_element_type=F32)
            + b2_ref[...]).astype(BF16)

    return pl.pallas_call(
        body,
        grid=(E // BLK,),
        in_specs=[
            pl.BlockSpec((BLK, H), lambda i: (i, 0)),
            pl.BlockSpec((BLK, 32), lambda i: (i, 0)),
            pl.BlockSpec((1, 1, BLK), lambda i: (i, 0, 0)),
            pl.BlockSpec((B, H), lambda i: (0, 0)),
            pl.BlockSpec((32, H), lambda i: (0, 0)),
            pl.BlockSpec((H, H), lambda i: (0, 0)),
            pl.BlockSpec((1, H), lambda i: (0, 0)),
            pl.BlockSpec((1, H), lambda i: (0, 0)),
        ],
        out_specs=pl.BlockSpec((BLK, H), lambda i: (i, 0)),
        out_shape=jax.ShapeDtypeStruct((E, H), BF16),
    )(G0, D, e2g3, latp, Wfp, W2, b1r, b2r)


def _sc_scatter(e2, src2, N2, E):
    H = 128
    EW = E // NW
    CH = EW // SUB
    STRIPE = N2 // 16
    mesh = plsc.VectorSubcoreMesh(core_axis_name="c", subcore_axis_name="s")

    @functools.partial(
        pl.kernel,
        mesh=mesh,
        out_type=(jax.ShapeDtypeStruct((2, N2, H), BF16),
                  jax.ShapeDtypeStruct((2, N2, 16), F32)),
        compiler_params=pltpu.CompilerParams(use_tc_tiling_on_sc=False),
        scratch_types=[
            pltpu.VMEM((CH, SUB), jnp.int32),
            pltpu.VMEM((SUB, H), BF16),
            pltpu.VMEM((SUB, H), BF16),
            pltpu.VMEM((SUB, 16), F32),
            pltpu.VMEM_SHARED((N2, H), BF16),
            pltpu.VMEM_SHARED((N2, 16), F32),
            pltpu.SemaphoreType.DMA,
            pltpu.SemaphoreType.DMA,
            pltpu.SemaphoreType.DMA,
            pltpu.SemaphoreType.DMA,
            pltpu.SemaphoreType.DMA,
            pltpu.SemaphoreType.DMA,
        ],
    )
    def k(e2_hbm, src_hbm, sum_out, cnt_out,
          idxs, ebuf0, ebuf1, onesb, acc_sh, cnt_sh,
          le0, le1, ss0, ss1, sc0, sc1):
        cid = lax.axis_index("c")
        sid = lax.axis_index("s")
        w32 = cid * 16 + sid
        pltpu.sync_copy(src_hbm.at[w32], idxs)

        ebuf = (ebuf0, ebuf1)
        le = (le0, le1)
        ss = (ss0, ss1)
        sc = (sc0, sc1)

        def fz(r, c):
            for cc in range(H // 32):
                ebuf0[r, pl.ds(cc * 32, 32)] = jnp.zeros((32,), BF16)
            onesb[r, :] = jnp.zeros((16,), F32)
            return c

        lax.fori_loop(0, SUB, fz, 0)

        def zc(t, c):
            pltpu.sync_copy(ebuf0,
                            acc_sh.at[pl.ds(sid * STRIPE + t * SUB, SUB)])
            pltpu.sync_copy(onesb,
                            cnt_sh.at[pl.ds(sid * STRIPE + t * SUB, SUB)])
            return c

        lax.fori_loop(0, STRIPE // SUB, zc, 0)

        def fo(r, c):
            onesb[r, :] = jnp.full((16,), 1.0, F32)
            return c

        lax.fori_loop(0, SUB, fo, 0)
        plsc.subcore_barrier()

        pltpu.async_copy(e2_hbm.at[pl.ds(w32 * EW, SUB)], ebuf0, le0)

        def pair(i, c):
            for b in range(2):
                j = 2 * i + b
                o = 1 - b

                @pl.when(j >= 1)
                def _():
                    pltpu.make_async_copy(
                        e2_hbm.at[pl.ds(0, SUB)], ebuf[o], ss[o]).wait()
                    pltpu.make_async_copy(
                        cnt_out.at[0, pl.ds(0, SUB)], onesb, sc[o]).wait()

                @pl.when(j + 1 < CH)
                def _():
                    pltpu.async_copy(
                        e2_hbm.at[pl.ds(w32 * EW + (j + 1) * SUB, SUB)],
                        ebuf[o], le[o])

                pltpu.make_async_copy(
                    e2_hbm.at[pl.ds(0, SUB)], ebuf[b], le[b]).wait()
                pltpu.async_copy(ebuf[b], acc_sh.at[idxs.at[j]], ss[b],
                                 add=True)
                pltpu.async_copy(onesb, cnt_sh.at[idxs.at[j]], sc[b],
                                 add=True)
            return c

        lax.fori_loop(0, CH // 2, pair, 0)
        pltpu.make_async_copy(e2_hbm.at[pl.ds(0, SUB)], ebuf1, ss1).wait()
        pltpu.make_async_copy(cnt_out.at[0, pl.ds(0, SUB)], onesb, sc1).wait()
        plsc.subcore_barrier()
        pltpu.sync_copy(acc_sh.at[pl.ds(sid * STRIPE, STRIPE)],
                        sum_out.at[cid, pl.ds(sid * STRIPE, STRIPE)])
        pltpu.sync_copy(cnt_sh.at[pl.ds(sid * STRIPE, STRIPE)],
                        cnt_out.at[cid, pl.ds(sid * STRIPE, STRIPE)])

    return k(e2, src2)


def _tc_node(nfp, p0, p1, c0, c1, Wn1a, Wn1b, Wn2, bn1r, bn2r):
    NP, H = nfp.shape
    BLK = 512

    def body(nf_ref, p0_ref, p1_ref, c0_ref, c1_ref,
             wa_ref, wb_ref, w2_ref, b1_ref, b2_ref, out_ref):
        nf = nf_ref[...]
        cnt = c0_ref[...][:, 0:1] + c1_ref[...][:, 0:1]
        agg = ((p0_ref[...].astype(F32) + p1_ref[...].astype(F32))
               / jnp.maximum(cnt, 1.0))
        h = (jnp.dot(nf, wa_ref[...], preferred_element_type=F32)
             + jnp.dot(agg, wb_ref[...], preferred_element_type=F32)
             + b1_ref[...])
        o = _silu(h)
        out_ref[...] = nf + _silu(
            jnp.dot(o, w2_ref[...], preferred_element_type=F32) + b2_ref[...])

    return pl.pallas_call(
        body,
        grid=(NP // BLK,),
        in_specs=[
            pl.BlockSpec((BLK, H), lambda i: (i, 0)),
            pl.BlockSpec((BLK, H), lambda i: (i, 0)),
            pl.BlockSpec((BLK, H), lambda i: (i, 0)),
            pl.BlockSpec((BLK, 16), lambda i: (i, 0)),
            pl.BlockSpec((BLK, 16), lambda i: (i, 0)),
            pl.BlockSpec((H, H), lambda i: (0, 0)),
            pl.BlockSpec((H, H), lambda i: (0, 0)),
            pl.BlockSpec((H, H), lambda i: (0, 0)),
            pl.BlockSpec((1, H), lambda i: (0, 0)),
            pl.BlockSpec((1, H), lambda i: (0, 0)),
        ],
        out_specs=pl.BlockSpec((BLK, H), lambda i: (i, 0)),
        out_shape=jax.ShapeDtypeStruct((NP, H), F32),
    )(nfp, p0, p1, c0, c1, Wn1a, Wn1b, Wn2, bn1r, bn2r)


def kernel(node_features, frac_coords, lattice_feats, edge_index, edge2graph,
           W_e1, b_e1, W_e2, b_e2, W_n1, b_n1, W_n2, b_n2):
    N, H = node_features.shape
    E = edge_index.shape[1]
    B = lattice_feats.shape[0]
    NP = ((N + 511) // 512) * 512

    nfp = jnp.pad(node_features, ((0, NP - N), (0, 0)))
    fcp = jnp.pad(frac_coords, ((0, NP - N), (0, 29)))
    W1a = W_e1[:H]
    W1b = W_e1[H:2 * H]
    W1cp = jnp.pad(W_e1[2 * H:2 * H + 9], ((0, 7), (0, 0)))
    Wfp = jnp.pad(W_e1[2 * H + 9:], ((0, 29), (0, 0)))
    lat9 = jnp.pad(lattice_feats.reshape(B, 9), ((0, 0), (0, 7)))

    TS, TD, latp = _tc_prep(nfp, fcp, W1a, W1b, lat9, W1cp)

    CH = E // NW // SUB
    src2 = edge_index[0].reshape(NW, CH, SUB)
    dst2 = edge_index[1].reshape(NW, CH, SUB)
    G0, D = _sc_gather(TS, TD, src2, dst2, E)

    e2g3 = edge2graph.reshape(E // 512, 1, 512)
    e2 = _tc_edge(G0, D, e2g3, latp, Wfp, W_e2,
                  b_e1.reshape(1, H), b_e2.reshape(1, H))

    sums, cnts = _sc_scatter(e2, src2, NP, E)

    out = _tc_node(nfp, sums[0], sums[1], cnts[0], cnts[1],
                   W_n1[:H], W_n1[H:], W_n2,
                   b_n1.reshape(1, H), b_n2.reshape(1, H))
    return out[:N]


# speedup vs baseline: 1.1587x; 1.1587x over previous
import functools
import jax
import jax.numpy as jnp
from jax import lax
from jax.experimental import pallas as pl
from jax.experimental.pallas import tpu as pltpu
from jax.experimental.pallas import tpu_sc as plsc

F32 = jnp.float32
BF16 = jnp.bfloat16
NW = 32
SUB = 40


def _silu(x):
    return x * jax.nn.sigmoid(x)


def _tc_prep(nfp, fcp, W1a, W1b, lat9, W1cp):
    NP, H = nfp.shape
    B = lat9.shape[0]
    BLK = 512
    WID = H + 32

    def body(nf_ref, fc_ref, wa_ref, wb_ref, l_ref, wc_ref,
             ts_ref, td_ref, lp_ref):
        nf = nf_ref[...]
        fc = fc_ref[...]
        ts_ref[:, :H] = jnp.dot(
            nf, wa_ref[...], preferred_element_type=F32).astype(BF16)
        ts_ref[:, H:] = (-fc).astype(BF16)
        td_ref[:, :H] = jnp.dot(
            nf, wb_ref[...], preferred_element_type=F32).astype(BF16)
        td_ref[:, H:] = fc.astype(BF16)
        l = l_ref[...]
        cols = []
        for i in range(3):
            for j in range(3):
                v = (l[:, 3 * i + 0:3 * i + 1] * l[:, 3 * j + 0:3 * j + 1]
                     + l[:, 3 * i + 1:3 * i + 2] * l[:, 3 * j + 1:3 * j + 2]
                     + l[:, 3 * i + 2:3 * i + 3] * l[:, 3 * j + 2:3 * j + 3])
                cols.append(v)
        cols.append(jnp.zeros_like(l[:, :7]))
        llt = jnp.concatenate(cols, axis=1)
        lp_ref[...] = jnp.dot(llt, wc_ref[...], preferred_element_type=F32)

    return pl.pallas_call(
        body,
        grid=(NP // BLK,),
        in_specs=[
            pl.BlockSpec((BLK, H), lambda i: (i, 0)),
            pl.BlockSpec((BLK, 32), lambda i: (i, 0)),
            pl.BlockSpec((H, H), lambda i: (0, 0)),
            pl.BlockSpec((H, H), lambda i: (0, 0)),
            pl.BlockSpec((B, 16), lambda i: (0, 0)),
            pl.BlockSpec((16, H), lambda i: (0, 0)),
        ],
        out_specs=[
            pl.BlockSpec((BLK, WID), lambda i: (i, 0)),
            pl.BlockSpec((BLK, WID), lambda i: (i, 0)),
            pl.BlockSpec((B, H), lambda i: (0, 0)),
        ],
        out_shape=[
            jax.ShapeDtypeStruct((NP, WID), BF16),
            jax.ShapeDtypeStruct((NP, WID), BF16),
            jax.ShapeDtypeStruct((B, H), F32),
        ],
    )(nfp, fcp, W1a, W1b, lat9, W1cp)


def _sc_gather(TS, TD, src2, dst2, E):
    W = TS.shape[1]
    EW = E // NW
    CH = EW // SUB
    mesh = plsc.VectorSubcoreMesh(core_axis_name="c", subcore_axis_name="s")

    H = 128

    @functools.partial(
        pl.kernel,
        mesh=mesh,
        out_type=(jax.ShapeDtypeStruct((E, H), BF16),
                  jax.ShapeDtypeStruct((E, 32), BF16)),
        compiler_params=pltpu.CompilerParams(use_tc_tiling_on_sc=False),
        scratch_types=[
            pltpu.VMEM((CH, SUB), jnp.int32),
            pltpu.VMEM((CH, SUB), jnp.int32),
            pltpu.VMEM((SUB, W), BF16),
            pltpu.VMEM((SUB, W), BF16),
            pltpu.VMEM((SUB, W), BF16),
            pltpu.VMEM((SUB, W), BF16),
            pltpu.VMEM((SUB, H), BF16),
            pltpu.VMEM((SUB, H), BF16),
            pltpu.VMEM((SUB, 32), BF16),
            pltpu.VMEM((SUB, 32), BF16),
            pltpu.SemaphoreType.DMA,
            pltpu.SemaphoreType.DMA,
            pltpu.SemaphoreType.DMA,
            pltpu.SemaphoreType.DMA,
            pltpu.SemaphoreType.DMA,
            pltpu.SemaphoreType.DMA,
            pltpu.SemaphoreType.DMA,
            pltpu.SemaphoreType.DMA,
        ],
    )
    def k(ts_hbm, td_hbm, src_hbm, dst_hbm, g0_hbm, d_hbm,
          idxs, idxd, bufa0, bufa1, bufb0, bufb1, gbuf0, gbuf1, dbuf0, dbuf1,
          la0, la1, lb0, lb1, sg0, sg1, sd0, sd1):
        cid = lax.axis_index("c")
        sid = lax.axis_index("s")
        w32 = cid * 16 + sid
        pltpu.sync_copy(src_hbm.at[w32], idxs)
        pltpu.sync_copy(dst_hbm.at[w32], idxd)

        bufa = (bufa0, bufa1)
        bufb = (bufb0, bufb1)
        gbuf = (gbuf0, gbuf1)
        dbuf = (dbuf0, dbuf1)
        la = (la0, la1)
        lb = (lb0, lb1)
        sg = (sg0, sg1)
        sd = (sd0, sd1)

        pltpu.async_copy(ts_hbm.at[idxs.at[0]], bufa0, la0)
        pltpu.async_copy(td_hbm.at[idxd.at[0]], bufb0, lb0)

        def pair(i, carry):
            for b in range(2):
                j = 2 * i + b
                o = 1 - b

                @pl.when(j + 1 < CH)
                def _():
                    pltpu.async_copy(ts_hbm.at[idxs.at[j + 1]], bufa[o], la[o])
                    pltpu.async_copy(td_hbm.at[idxd.at[j + 1]], bufb[o], lb[o])

                pltpu.make_async_copy(
                    ts_hbm.at[pl.ds(0, SUB)], bufa[b], la[b]).wait()
                pltpu.make_async_copy(
                    td_hbm.at[pl.ds(0, SUB)], bufb[b], lb[b]).wait()

                @pl.when(j >= 2)
                def _():
                    pltpu.make_async_copy(
                        g0_hbm.at[pl.ds(0, SUB)], gbuf[b], sg[b]).wait()
                    pltpu.make_async_copy(
                        d_hbm.at[pl.ds(0, SUB)], dbuf[b], sd[b]).wait()

                def radd(r, c2, _b=b):
                    for cc in range(H // 32):
                        sl = pl.ds(cc * 32, 32)
                        gbuf[_b][r, sl] = bufa[_b][r, sl] + bufb[_b][r, sl]
                    sl = pl.ds(H, 32)
                    dbuf[_b][r, :] = bufa[_b][r, sl] + bufb[_b][r, sl]
                    return c2

                lax.fori_loop(0, SUB, radd, 0)
                rows = pl.ds(w32 * EW + j * SUB, SUB)
                pltpu.async_copy(gbuf[b], g0_hbm.at[rows], sg[b])
                pltpu.async_copy(dbuf[b], d_hbm.at[rows], sd[b])
            return carry

        lax.fori_loop(0, CH // 2, pair, 0)
        for b in range(2):
            pltpu.make_async_copy(
                g0_hbm.at[pl.ds(0, SUB)], gbuf[b], sg[b]).wait()
            pltpu.make_async_copy(
                d_hbm.at[pl.ds(0, SUB)], dbuf[b], sd[b]).wait()

    return k(TS, TD, src2, dst2)


def _tc_edge(G0, D, e2g3, latp, Wfp, W2, b1r, b2r):
    E, H = G0.shape
    B = latp.shape[0]
    BLK = 512

    def body(g_ref, d_ref, id_ref, lp_ref, wf_ref, w2_ref, b1_ref, b2_ref,
             out_ref):
        g = g_ref[...].astype(F32)
        d = d_ref[...].astype(F32)
        fd = d - jnp.floor(d)
        ids = id_ref[0, 0, :]
        oh = (ids[:, None] == lax.broadcasted_iota(jnp.int32, (BLK, B), 1)
              ).astype(F32)
        pre = (g
               + jnp.dot(oh, lp_ref[...], preferred_element_type=F32)
               + jnp.dot(fd, wf_ref[...], preferred_element_type=F32)
               + b1_ref[...])
        e1 = _silu(pre)
        out_ref[...] = _silu(
            jnp.dot(e1, w2_ref[...], preferred_element_type=F32)
            + b2_ref[...])

    return pl.pallas_call(
        body,
        grid=(E // BLK,),
        in_specs=[
            pl.BlockSpec((BLK, H), lambda i: (i, 0)),
            pl.BlockSpec((BLK, 32), lambda i: (i, 0)),
            pl.BlockSpec((1, 1, BLK), lambda i: (i, 0, 0)),
            pl.BlockSpec((B, H), lambda i: (0, 0)),
            pl.BlockSpec((32, H), lambda i: (0, 0)),
            pl.BlockSpec((H, H), lambda i: (0, 0)),
            pl.BlockSpec((1, H), lambda i: (0, 0)),
            pl.BlockSpec((1, H), lambda i: (0, 0)),
        ],
        out_specs=pl.BlockSpec((BLK, H), lambda i: (i, 0)),
        out_shape=jax.ShapeDtypeStruct((E, H), F32),
    )(G0, D, e2g3, latp, Wfp, W2, b1r, b2r)


def _sc_scatter(e2, src2, N2, E):
    H = 128
    EW = E // NW
    CH = EW // SUB
    STRIPE = N2 // 16
    mesh = plsc.VectorSubcoreMesh(core_axis_name="c", subcore_axis_name="s")

    @functools.partial(
        pl.kernel,
        mesh=mesh,
        out_type=(jax.ShapeDtypeStruct((2, N2, H), F32),
                  jax.ShapeDtypeStruct((2, N2, 16), F32)),
        compiler_params=pltpu.CompilerParams(use_tc_tiling_on_sc=False),
        scratch_types=[
            pltpu.VMEM((CH, SUB), jnp.int32),
            pltpu.VMEM((SUB, H), F32),
            pltpu.VMEM((SUB, H), F32),
            pltpu.VMEM((SUB, 16), F32),
            pltpu.VMEM_SHARED((N2, H), F32),
            pltpu.VMEM_SHARED((N2, 16), F32),
            pltpu.SemaphoreType.DMA,
            pltpu.SemaphoreType.DMA,
            pltpu.SemaphoreType.DMA,
            pltpu.SemaphoreType.DMA,
            pltpu.SemaphoreType.DMA,
            pltpu.SemaphoreType.DMA,
        ],
    )
    def k(e2_hbm, src_hbm, sum_out, cnt_out,
          idxs, ebuf0, ebuf1, onesb, acc_sh, cnt_sh,
          le0, le1, ss0, ss1, sc0, sc1):
        cid = lax.axis_index("c")
        sid = lax.axis_index("s")
        w32 = cid * 16 + sid
        pltpu.sync_copy(src_hbm.at[w32], idxs)

        ebuf = (ebuf0, ebuf1)
        le = (le0, le1)
        ss = (ss0, ss1)
        sc = (sc0, sc1)

        def fz(r, c):
            for cc in range(H // 16):
                ebuf0[r, pl.ds(cc * 16, 16)] = jnp.zeros((16,), F32)
            onesb[r, :] = jnp.zeros((16,), F32)
            return c

        lax.fori_loop(0, SUB, fz, 0)

        def zc(t, c):
            pltpu.sync_copy(ebuf0,
                            acc_sh.at[pl.ds(sid * STRIPE + t * SUB, SUB)])
            pltpu.sync_copy(onesb,
                            cnt_sh.at[pl.ds(sid * STRIPE + t * SUB, SUB)])
            return c

        lax.fori_loop(0, STRIPE // SUB, zc, 0)

        def fo(r, c):
            onesb[r, :] = jnp.full((16,), 1.0, F32)
            return c

        lax.fori_loop(0, SUB, fo, 0)
        plsc.subcore_barrier()

        pltpu.async_copy(e2_hbm.at[pl.ds(w32 * EW, SUB)], ebuf0, le0)

        def pair(i, c):
            for b in range(2):
                j = 2 * i + b
                o = 1 - b

                @pl.when(j >= 1)
                def _():
                    pltpu.make_async_copy(
                        e2_hbm.at[pl.ds(0, SUB)], ebuf[o], ss[o]).wait()
                    pltpu.make_async_copy(
                        cnt_out.at[0, pl.ds(0, SUB)], onesb, sc[o]).wait()

                @pl.when(j + 1 < CH)
                def _():
                    pltpu.async_copy(
                        e2_hbm.at[pl.ds(w32 * EW + (j + 1) * SUB, SUB)],
                        ebuf[o], le[o])

                pltpu.make_async_copy(
                    e2_hbm.at[pl.ds(0, SUB)], ebuf[b], le[b]).wait()
                pltpu.async_copy(ebuf[b], acc_sh.at[idxs.at[j]], ss[b],
                                 add=True)
                pltpu.async_copy(onesb, cnt_sh.at[idxs.at[j]], sc[b],
                                 add=True)
            return c

        lax.fori_loop(0, CH // 2, pair, 0)
        pltpu.make_async_copy(e2_hbm.at[pl.ds(0, SUB)], ebuf1, ss1).wait()
        pltpu.make_async_copy(cnt_out.at[0, pl.ds(0, SUB)], onesb, sc1).wait()
        plsc.subcore_barrier()
        pltpu.sync_copy(acc_sh.at[pl.ds(sid * STRIPE, STRIPE)],
                        sum_out.at[cid, pl.ds(sid * STRIPE, STRIPE)])
        pltpu.sync_copy(cnt_sh.at[pl.ds(sid * STRIPE, STRIPE)],
                        cnt_out.at[cid, pl.ds(sid * STRIPE, STRIPE)])

    return k(e2, src2)


def _tc_node(nfp, p0, p1, c0, c1, Wn1a, Wn1b, Wn2, bn1r, bn2r):
    NP, H = nfp.shape
    BLK = 512

    def body(nf_ref, p0_ref, p1_ref, c0_ref, c1_ref,
             wa_ref, wb_ref, w2_ref, b1_ref, b2_ref, out_ref):
        nf = nf_ref[...]
        cnt = c0_ref[...][:, 0:1] + c1_ref[...][:, 0:1]
        agg = (p0_ref[...] + p1_ref[...]) / jnp.maximum(cnt, 1.0)
        h = (jnp.dot(nf, wa_ref[...], preferred_element_type=F32)
             + jnp.dot(agg, wb_ref[...], preferred_element_type=F32)
             + b1_ref[...])
        o = _silu(h)
        out_ref[...] = nf + _silu(
            jnp.dot(o, w2_ref[...], preferred_element_type=F32) + b2_ref[...])

    return pl.pallas_call(
        body,
        grid=(NP // BLK,),
        in_specs=[
            pl.BlockSpec((BLK, H), lambda i: (i, 0)),
            pl.BlockSpec((BLK, H), lambda i: (i, 0)),
            pl.BlockSpec((BLK, H), lambda i: (i, 0)),
            pl.BlockSpec((BLK, 16), lambda i: (i, 0)),
            pl.BlockSpec((BLK, 16), lambda i: (i, 0)),
            pl.BlockSpec((H, H), lambda i: (0, 0)),
            pl.BlockSpec((H, H), lambda i: (0, 0)),
            pl.BlockSpec((H, H), lambda i: (0, 0)),
            pl.BlockSpec((1, H), lambda i: (0, 0)),
            pl.BlockSpec((1, H), lambda i: (0, 0)),
        ],
        out_specs=pl.BlockSpec((BLK, H), lambda i: (i, 0)),
        out_shape=jax.ShapeDtypeStruct((NP, H), F32),
    )(nfp, p0, p1, c0, c1, Wn1a, Wn1b, Wn2, bn1r, bn2r)


def kernel(node_features, frac_coords, lattice_feats, edge_index, edge2graph,
           W_e1, b_e1, W_e2, b_e2, W_n1, b_n1, W_n2, b_n2):
    N, H = node_features.shape
    E = edge_index.shape[1]
    B = lattice_feats.shape[0]
    NP = ((N + 511) // 512) * 512

    nfp = jnp.pad(node_features, ((0, NP - N), (0, 0)))
    fcp = jnp.pad(frac_coords, ((0, NP - N), (0, 29)))
    W1a = W_e1[:H]
    W1b = W_e1[H:2 * H]
    W1cp = jnp.pad(W_e1[2 * H:2 * H + 9], ((0, 7), (0, 0)))
    Wfp = jnp.pad(W_e1[2 * H + 9:], ((0, 29), (0, 0)))
    lat9 = jnp.pad(lattice_feats.reshape(B, 9), ((0, 0), (0, 7)))

    TS, TD, latp = _tc_prep(nfp, fcp, W1a, W1b, lat9, W1cp)

    CH = E // NW // SUB
    src2 = edge_index[0].reshape(NW, CH, SUB)
    dst2 = edge_index[1].reshape(NW, CH, SUB)
    G0, D = _sc_gather(TS, TD, src2, dst2, E)

    e2g3 = edge2graph.reshape(E // 512, 1, 512)
    e2 = _tc_edge(G0, D, e2g3, latp, Wfp, W_e2,
                  b_e1.reshape(1, H), b_e2.reshape(1, H))

    sums, cnts = _sc_scatter(e2, src2, NP, E)

    out = _tc_node(nfp, sums[0], sums[1], cnts[0], cnts[1],
                   W_n1[:H], W_n1[H:], W_n2,
                   b_n1.reshape(1, H), b_n2.reshape(1, H))
    return out[:N]
